# initial kernel scaffold (unmeasured)
import jax
import jax.numpy as jnp
from jax import lax
from jax.experimental import pallas as pl
from jax.experimental.pallas import tpu as pltpu

N_DEV = 4


def kernel(x, w_mat, scale_x, scale_w):
    m_per, k = x.shape
    n_total = w_mat.shape[1]
    n_per = n_total // N_DEV

    my = lax.axis_index("i")
    w_loc = lax.dynamic_slice_in_dim(w_mat, my * n_per, n_per, axis=1)
    s = (scale_x[0] * scale_w[0]).reshape(1, 1)

    def body(x_ref, w_ref, s_ref, out_ref, comm_ref, send_sems, recv_sems):
        my_pos = lax.axis_index("i")
        left = (my_pos - 1) % N_DEV
        right = (my_pos + 1) % N_DEV

        barrier_sem = pltpu.get_barrier_semaphore()
        for nbr in [left, right]:
            pl.semaphore_signal(
                barrier_sem, inc=1,
                device_id=(nbr,), device_id_type=pl.DeviceIdType.MESH,
            )
        pl.semaphore_wait(barrier_sem, 2)

        scale = s_ref[0, 0]

        def compute(origin, chunk):
            acc = lax.dot_general(
                chunk, w_ref[...],
                (((1,), (0,)), ((), ())),
                preferred_element_type=jnp.int32,
            )
            y = jnp.maximum(acc.astype(jnp.float32) * scale, 0.0)
            out_ref[pl.ds(origin * m_per, m_per), :] = y

        comm_ref[0] = x_ref[...]

        for h in range(N_DEV - 1):
            rdma = pltpu.make_async_remote_copy(
                src_ref=comm_ref.at[h],
                dst_ref=comm_ref.at[h + 1],
                send_sem=send_sems.at[h],
                recv_sem=recv_sems.at[h],
                device_id=(right,),
                device_id_type=pl.DeviceIdType.MESH,
            )
            rdma.start()
            compute((my_pos - h) % N_DEV, comm_ref[h])
            rdma.wait()

        compute((my_pos - (N_DEV - 1)) % N_DEV, comm_ref[N_DEV - 1])

    return pl.pallas_call(
        body,
        out_shape=jax.ShapeDtypeStruct((N_DEV * m_per, n_per), jnp.float32),
        in_specs=[
            pl.BlockSpec(memory_space=pltpu.VMEM),
            pl.BlockSpec(memory_space=pltpu.VMEM),
            pl.BlockSpec(memory_space=pltpu.SMEM),
        ],
        out_specs=pl.BlockSpec(memory_space=pltpu.VMEM),
        scratch_shapes=[
            pltpu.VMEM((N_DEV, m_per, k), x.dtype),
            pltpu.SemaphoreType.DMA((N_DEV - 1,)),
            pltpu.SemaphoreType.DMA((N_DEV - 1,)),
        ],
        compiler_params=pltpu.CompilerParams(collective_id=0),
    )(x, w_loc, s)


# baseline (device time: 206316 ns/iter reference)
import jax
import jax.numpy as jnp
from jax import lax
from jax.experimental import pallas as pl
from jax.experimental.pallas import tpu as pltpu

N_DEV = 4
N_TILES = 2


def kernel(x, w_mat, scale_x, scale_w):
    m_per, k = x.shape
    n_total = w_mat.shape[1]
    n_per = n_total // N_DEV
    n_tile = n_per // N_TILES

    my = lax.axis_index("i")
    w_loc = lax.dynamic_slice_in_dim(w_mat, my * n_per, n_per, axis=1)
    s = (scale_x[0] * scale_w[0]).reshape(1, 1)

    def body(x_ref, w_ref, s_ref, out_ref,
             comm_ref, y_ref, send_sems, recv_sems, copy_sems):
        my_pos = lax.axis_index("i")
        left = (my_pos - 1) % N_DEV
        right = (my_pos + 1) % N_DEV

        barrier_sem = pltpu.get_barrier_semaphore()
        for nbr in [left, right]:
            pl.semaphore_signal(
                barrier_sem, inc=1,
                device_id=(nbr,), device_id_type=pl.DeviceIdType.MESH,
            )
        pl.semaphore_wait(barrier_sem, 2)

        scale = s_ref[0, 0]
        copies = [None] * N_DEV

        def compute(c, origin, chunk):
            slot = c % 2
            if c >= 2:
                copies[c - 2].wait()
            for j in range(N_TILES):
                acc = lax.dot_general(
                    chunk, w_ref[:, j * n_tile:(j + 1) * n_tile],
                    (((1,), (0,)), ((), ())),
                    preferred_element_type=jnp.int32,
                )
                y_ref[slot, :, j * n_tile:(j + 1) * n_tile] = jnp.maximum(
                    acc.astype(jnp.float32) * scale, 0.0)
            cp = pltpu.make_async_copy(
                y_ref.at[slot],
                out_ref.at[pl.ds(origin * m_per, m_per), :],
                copy_sems.at[c],
            )
            cp.start()
            copies[c] = cp

        comm_ref[0] = x_ref[...]

        for h in range(N_DEV - 1):
            rdma = pltpu.make_async_remote_copy(
                src_ref=comm_ref.at[h],
                dst_ref=comm_ref.at[h + 1],
                send_sem=send_sems.at[h],
                recv_sem=recv_sems.at[h],
                device_id=(right,),
                device_id_type=pl.DeviceIdType.MESH,
            )
            rdma.start()
            compute(h, (my_pos - h) % N_DEV, comm_ref[h])
            rdma.wait()

        compute(N_DEV - 1, (my_pos - (N_DEV - 1)) % N_DEV,
                comm_ref[N_DEV - 1])
        copies[N_DEV - 2].wait()
        copies[N_DEV - 1].wait()

    return pl.pallas_call(
        body,
        out_shape=jax.ShapeDtypeStruct((N_DEV * m_per, n_per), jnp.float32),
        in_specs=[
            pl.BlockSpec(memory_space=pltpu.VMEM),
            pl.BlockSpec(memory_space=pltpu.VMEM),
            pl.BlockSpec(memory_space=pltpu.SMEM),
        ],
        out_specs=pl.BlockSpec(memory_space=pl.ANY),
        scratch_shapes=[
            pltpu.VMEM((N_DEV, m_per, k), x.dtype),
            pltpu.VMEM((2, m_per, n_per), jnp.float32),
            pltpu.SemaphoreType.DMA((N_DEV - 1,)),
            pltpu.SemaphoreType.DMA((N_DEV - 1,)),
            pltpu.SemaphoreType.DMA((N_DEV,)),
        ],
        compiler_params=pltpu.CompilerParams(
            collective_id=0,
            vmem_limit_bytes=60 * 1024 * 1024,
        ),
    )(x, w_loc, s)


# device time: 157837 ns/iter; 1.3071x vs baseline; 1.3071x over previous
import jax
import jax.numpy as jnp
from jax import lax
from jax.experimental import pallas as pl
from jax.experimental.pallas import tpu as pltpu

N_DEV = 4
N_TILES = 2

OWN, FROM_L, FROM_R, DIAG = 0, 1, 2, 3
S_OWN_R, S_OWN_L, S_REL_R, S_REL_L = 0, 1, 2, 3


def kernel(x, w_mat, scale_x, scale_w):
    m_per, k = x.shape
    half = m_per // 2
    n_total = w_mat.shape[1]
    n_per = n_total // N_DEV
    n_tile = n_per // N_TILES

    my = lax.axis_index("i")
    w_loc = lax.dynamic_slice_in_dim(w_mat, my * n_per, n_per, axis=1)
    s = (scale_x[0] * scale_w[0]).reshape(1, 1)

    def body(x_ref, w_ref, s_ref, out_ref,
             comm_ref, y_ref, send_sems, recv_sems, copy_sems):
        my_pos = lax.axis_index("i")
        left = (my_pos - 1) % N_DEV
        right = (my_pos + 1) % N_DEV

        barrier_sem = pltpu.get_barrier_semaphore()
        for nbr in [left, right]:
            pl.semaphore_signal(
                barrier_sem, inc=1,
                device_id=(nbr,), device_id_type=pl.DeviceIdType.MESH,
            )
        pl.semaphore_wait(barrier_sem, 2)

        scale = s_ref[0, 0]
        copies = [None] * N_DEV

        def compute(c, origin, chunk):
            slot = c % 2
            if c >= 2:
                copies[c - 2].wait()
            for j in range(N_TILES):
                acc = lax.dot_general(
                    chunk, w_ref[:, j * n_tile:(j + 1) * n_tile],
                    (((1,), (0,)), ((), ())),
                    preferred_element_type=jnp.int32,
                )
                y_ref[slot, :, j * n_tile:(j + 1) * n_tile] = jnp.maximum(
                    acc.astype(jnp.float32) * scale, 0.0)
            cp = pltpu.make_async_copy(
                y_ref.at[slot],
                out_ref.at[pl.ds(origin * m_per, m_per), :],
                copy_sems.at[c],
            )
            cp.start()
            copies[c] = cp

        def xfer(src, dst, sem_idx, target):
            return pltpu.make_async_remote_copy(
                src_ref=src, dst_ref=dst,
                send_sem=send_sems.at[sem_idx],
                recv_sem=recv_sems.at[sem_idx],
                device_id=(target,),
                device_id_type=pl.DeviceIdType.MESH,
            )

        comm_ref[OWN] = x_ref[...]

        send_r = xfer(comm_ref.at[OWN], comm_ref.at[FROM_L], S_OWN_R, right)
        send_l = xfer(comm_ref.at[OWN], comm_ref.at[FROM_R], S_OWN_L, left)
        send_r.start()
        send_l.start()

        compute(0, my_pos, comm_ref[OWN])

        send_r.wait_recv()
        relay_r = xfer(comm_ref.at[FROM_L, pl.ds(0, half)],
                       comm_ref.at[DIAG, pl.ds(0, half)], S_REL_R, right)
        relay_r.start()
        compute(1, left, comm_ref[FROM_L])

        send_l.wait_recv()
        relay_l = xfer(comm_ref.at[FROM_R, pl.ds(half, half)],
                       comm_ref.at[DIAG, pl.ds(half, half)], S_REL_L, left)
        relay_l.start()
        compute(2, right, comm_ref[FROM_R])

        relay_r.wait_recv()
        relay_l.wait_recv()
        compute(3, (my_pos + 2) % N_DEV, comm_ref[DIAG])

        send_r.wait_send()
        send_l.wait_send()
        relay_r.wait_send()
        relay_l.wait_send()
        copies[N_DEV - 2].wait()
        copies[N_DEV - 1].wait()

    return pl.pallas_call(
        body,
        out_shape=jax.ShapeDtypeStruct((N_DEV * m_per, n_per), jnp.float32),
        in_specs=[
            pl.BlockSpec(memory_space=pltpu.VMEM),
            pl.BlockSpec(memory_space=pltpu.VMEM),
            pl.BlockSpec(memory_space=pltpu.SMEM),
        ],
        out_specs=pl.BlockSpec(memory_space=pl.ANY),
        scratch_shapes=[
            pltpu.VMEM((N_DEV, m_per, k), x.dtype),
            pltpu.VMEM((2, m_per, n_per), jnp.float32),
            pltpu.SemaphoreType.DMA((N_DEV,)),
            pltpu.SemaphoreType.DMA((N_DEV,)),
            pltpu.SemaphoreType.DMA((N_DEV,)),
        ],
        compiler_params=pltpu.CompilerParams(
            collective_id=0,
            vmem_limit_bytes=60 * 1024 * 1024,
        ),
    )(x, w_loc, s)


# device time: 132547 ns/iter; 1.5565x vs baseline; 1.1908x over previous
import jax
import jax.numpy as jnp
from jax import lax
from jax.experimental import pallas as pl
from jax.experimental.pallas import tpu as pltpu

N_DEV = 4
N_TILES = 2

OWN, FROM_L, FROM_R, DIAG = 0, 1, 2, 3
S_OWN_R, S_OWN_L, S_REL_R, S_REL_L = 0, 1, 2, 3


def kernel(x, w_mat, scale_x, scale_w):
    m_per, k = x.shape
    half = m_per // 2
    n_total = w_mat.shape[1]
    n_per = n_total // N_DEV
    n_tile = n_per // N_TILES

    s = (scale_x[0] * scale_w[0]).reshape(1, 1)

    def body(x_hbm, w_hbm, s_ref, out_ref,
             comm_ref, w_ref, y_ref, send_sems, recv_sems, copy_sems,
             in_sems):
        my_pos = lax.axis_index("i")
        left = (my_pos - 1) % N_DEV
        right = (my_pos + 1) % N_DEV

        x_dma = pltpu.make_async_copy(x_hbm, comm_ref.at[OWN],
                                      in_sems.at[0])
        x_dma.start()
        w_dma = pltpu.make_async_copy(
            w_hbm.at[:, pl.ds(my_pos * n_per, n_per)], w_ref, in_sems.at[1])
        w_dma.start()

        barrier_sem = pltpu.get_barrier_semaphore()
        for nbr in [left, right]:
            pl.semaphore_signal(
                barrier_sem, inc=1,
                device_id=(nbr,), device_id_type=pl.DeviceIdType.MESH,
            )
        pl.semaphore_wait(barrier_sem, 2)

        scale = s_ref[0, 0]
        copies = [None] * N_DEV

        def compute(c, origin, chunk):
            slot = c % 2
            if c >= 2:
                copies[c - 2].wait()
            for j in range(N_TILES):
                acc = lax.dot_general(
                    chunk, w_ref[:, j * n_tile:(j + 1) * n_tile],
                    (((1,), (0,)), ((), ())),
                    preferred_element_type=jnp.int32,
                )
                y_ref[slot, :, j * n_tile:(j + 1) * n_tile] = jnp.maximum(
                    acc.astype(jnp.float32) * scale, 0.0)
            cp = pltpu.make_async_copy(
                y_ref.at[slot],
                out_ref.at[pl.ds(origin * m_per, m_per), :],
                copy_sems.at[c],
            )
            cp.start()
            copies[c] = cp

        def xfer(src, dst, sem_idx, target):
            return pltpu.make_async_remote_copy(
                src_ref=src, dst_ref=dst,
                send_sem=send_sems.at[sem_idx],
                recv_sem=recv_sems.at[sem_idx],
                device_id=(target,),
                device_id_type=pl.DeviceIdType.MESH,
            )

        x_dma.wait()
        send_r = xfer(comm_ref.at[OWN], comm_ref.at[FROM_L], S_OWN_R, right)
        send_l = xfer(comm_ref.at[OWN], comm_ref.at[FROM_R], S_OWN_L, left)
        send_r.start()
        send_l.start()

        w_dma.wait()
        compute(0, my_pos, comm_ref[OWN])

        send_r.wait_recv()
        relay_r = xfer(comm_ref.at[FROM_L, pl.ds(0, half)],
                       comm_ref.at[DIAG, pl.ds(0, half)], S_REL_R, right)
        relay_r.start()
        send_l.wait_recv()
        relay_l = xfer(comm_ref.at[FROM_R, pl.ds(half, half)],
                       comm_ref.at[DIAG, pl.ds(half, half)], S_REL_L, left)
        relay_l.start()

        compute(1, left, comm_ref[FROM_L])
        compute(2, right, comm_ref[FROM_R])

        relay_r.wait_recv()
        relay_l.wait_recv()
        compute(3, (my_pos + 2) % N_DEV, comm_ref[DIAG])

        send_r.wait_send()
        send_l.wait_send()
        relay_r.wait_send()
        relay_l.wait_send()
        copies[N_DEV - 2].wait()
        copies[N_DEV - 1].wait()

    return pl.pallas_call(
        body,
        out_shape=jax.ShapeDtypeStruct((N_DEV * m_per, n_per), jnp.float32),
        in_specs=[
            pl.BlockSpec(memory_space=pl.ANY),
            pl.BlockSpec(memory_space=pl.ANY),
            pl.BlockSpec(memory_space=pltpu.SMEM),
        ],
        out_specs=pl.BlockSpec(memory_space=pl.ANY),
        scratch_shapes=[
            pltpu.VMEM((N_DEV, m_per, k), x.dtype),
            pltpu.VMEM((k, n_per), w_mat.dtype),
            pltpu.VMEM((2, m_per, n_per), jnp.float32),
            pltpu.SemaphoreType.DMA((N_DEV,)),
            pltpu.SemaphoreType.DMA((N_DEV,)),
            pltpu.SemaphoreType.DMA((N_DEV,)),
            pltpu.SemaphoreType.DMA((2,)),
        ],
        compiler_params=pltpu.CompilerParams(
            collective_id=0,
            vmem_limit_bytes=60 * 1024 * 1024,
        ),
    )(x, w_mat, s)


# device time: 119858 ns/iter; 1.7213x vs baseline; 1.1059x over previous
import jax
import jax.numpy as jnp
from jax import lax
from jax.experimental import pallas as pl
from jax.experimental.pallas import tpu as pltpu

N_DEV = 4
N_TILES = 2
N_YSLOTS = 4

OWN, FROM_L, FROM_R, DIAG = 0, 1, 2, 3
S_R_H1, S_R_H2, S_L_H1, S_L_H2, S_REL_R, S_REL_L = range(6)


def kernel(x, w_mat, scale_x, scale_w):
    m_per, k = x.shape
    half = m_per // 2
    n_total = w_mat.shape[1]
    n_per = n_total // N_DEV
    n_tile = n_per // N_TILES

    s = (scale_x[0] * scale_w[0]).reshape(1, 1)

    def body(x_hbm, w_hbm, s_ref, out_ref,
             comm_ref, w_ref, y_ref, send_sems, recv_sems, copy_sems,
             in_sems):
        my_pos = lax.axis_index("i")
        left = (my_pos - 1) % N_DEV
        right = (my_pos + 1) % N_DEV

        x_dma = pltpu.make_async_copy(x_hbm, comm_ref.at[OWN],
                                      in_sems.at[0])
        x_dma.start()
        w_dma = pltpu.make_async_copy(
            w_hbm.at[:, pl.ds(my_pos * n_per, n_per)], w_ref, in_sems.at[1])
        w_dma.start()

        barrier_sem = pltpu.get_barrier_semaphore()
        for nbr in [left, right]:
            pl.semaphore_signal(
                barrier_sem, inc=1,
                device_id=(nbr,), device_id_type=pl.DeviceIdType.MESH,
            )
        pl.semaphore_wait(barrier_sem, 2)

        scale = s_ref[0, 0]
        copies = [None] * (2 * N_DEV)
        unit = [0]

        def compute_half(origin, chunk_half, row_half):
            c = unit[0]
            unit[0] += 1
            slot = c % N_YSLOTS
            if c >= N_YSLOTS:
                copies[c - N_YSLOTS].wait()
            for j in range(N_TILES):
                acc = lax.dot_general(
                    chunk_half, w_ref[:, j * n_tile:(j + 1) * n_tile],
                    (((1,), (0,)), ((), ())),
                    preferred_element_type=jnp.int32,
                )
                y_ref[slot, :, j * n_tile:(j + 1) * n_tile] = jnp.maximum(
                    acc.astype(jnp.float32) * scale, 0.0)
            cp = pltpu.make_async_copy(
                y_ref.at[slot],
                out_ref.at[pl.ds(origin * m_per + row_half * half, half), :],
                copy_sems.at[c],
            )
            cp.start()
            copies[c] = cp

        def xfer(src, dst, sem_idx, target):
            return pltpu.make_async_remote_copy(
                src_ref=src, dst_ref=dst,
                send_sem=send_sems.at[sem_idx],
                recv_sem=recv_sems.at[sem_idx],
                device_id=(target,),
                device_id_type=pl.DeviceIdType.MESH,
            )

        x_dma.wait()

        send_r_h1 = xfer(comm_ref.at[OWN, pl.ds(0, half)],
                         comm_ref.at[FROM_L, pl.ds(0, half)], S_R_H1, right)
        send_l_h2 = xfer(comm_ref.at[OWN, pl.ds(half, half)],
                         comm_ref.at[FROM_R, pl.ds(half, half)], S_L_H2, left)
        send_r_h2 = xfer(comm_ref.at[OWN, pl.ds(half, half)],
                         comm_ref.at[FROM_L, pl.ds(half, half)], S_R_H2, right)
        send_l_h1 = xfer(comm_ref.at[OWN, pl.ds(0, half)],
                         comm_ref.at[FROM_R, pl.ds(0, half)], S_L_H1, left)
        send_r_h1.start()
        send_l_h2.start()
        send_r_h2.start()
        send_l_h1.start()

        w_dma.wait()
        compute_half(my_pos, comm_ref[OWN, :half], 0)
        compute_half(my_pos, comm_ref[OWN, half:], 1)

        send_r_h1.wait_recv()
        relay_r = xfer(comm_ref.at[FROM_L, pl.ds(0, half)],
                       comm_ref.at[DIAG, pl.ds(0, half)], S_REL_R, right)
        relay_r.start()
        compute_half(left, comm_ref[FROM_L, :half], 0)

        send_l_h2.wait_recv()
        relay_l = xfer(comm_ref.at[FROM_R, pl.ds(half, half)],
                       comm_ref.at[DIAG, pl.ds(half, half)], S_REL_L, left)
        relay_l.start()
        compute_half(right, comm_ref[FROM_R, half:], 1)

        send_r_h2.wait_recv()
        compute_half(left, comm_ref[FROM_L, half:], 1)
        send_l_h1.wait_recv()
        compute_half(right, comm_ref[FROM_R, :half], 0)

        diag = (my_pos + 2) % N_DEV
        relay_r.wait_recv()
        compute_half(diag, comm_ref[DIAG, :half], 0)
        relay_l.wait_recv()
        compute_half(diag, comm_ref[DIAG, half:], 1)

        for rdma in (send_r_h1, send_r_h2, send_l_h1, send_l_h2,
                     relay_r, relay_l):
            rdma.wait_send()
        for c in range(2 * N_DEV - N_YSLOTS, 2 * N_DEV):
            copies[c].wait()

    return pl.pallas_call(
        body,
        out_shape=jax.ShapeDtypeStruct((N_DEV * m_per, n_per), jnp.float32),
        in_specs=[
            pl.BlockSpec(memory_space=pl.ANY),
            pl.BlockSpec(memory_space=pl.ANY),
            pl.BlockSpec(memory_space=pltpu.SMEM),
        ],
        out_specs=pl.BlockSpec(memory_space=pl.ANY),
        scratch_shapes=[
            pltpu.VMEM((N_DEV, m_per, k), x.dtype),
            pltpu.VMEM((k, n_per), w_mat.dtype),
            pltpu.VMEM((N_YSLOTS, half, n_per), jnp.float32),
            pltpu.SemaphoreType.DMA((6,)),
            pltpu.SemaphoreType.DMA((6,)),
            pltpu.SemaphoreType.DMA((2 * N_DEV,)),
            pltpu.SemaphoreType.DMA((2,)),
        ],
        compiler_params=pltpu.CompilerParams(
            collective_id=0,
            vmem_limit_bytes=60 * 1024 * 1024,
        ),
    )(x, w_mat, s)


# device time: 107104 ns/iter; 1.9263x vs baseline; 1.1191x over previous
import jax
import jax.numpy as jnp
from jax import lax
from jax.experimental import pallas as pl
from jax.experimental.pallas import tpu as pltpu

N_DEV = 4
N_TILES = 2
N_Q = 4
N_YSLOTS = 4

OWN, FROM_L, FROM_R, DIAG = 0, 1, 2, 3


def kernel(x, w_mat, scale_x, scale_w):
    m_per, k = x.shape
    q = m_per // N_Q
    n_total = w_mat.shape[1]
    n_per = n_total // N_DEV
    n_tile = n_per // N_TILES

    s = (scale_x[0] * scale_w[0]).reshape(1, 1)

    def body(x_hbm, w_hbm, s_ref, out_ref,
             comm_ref, w_ref, y_ref, send_sems, recv_sems, copy_sems,
             in_sems):
        my_pos = lax.axis_index("i")
        left = (my_pos - 1) % N_DEV
        right = (my_pos + 1) % N_DEV

        x_dma = pltpu.make_async_copy(x_hbm, comm_ref.at[OWN],
                                      in_sems.at[0])
        x_dma.start()
        w_dma = pltpu.make_async_copy(
            w_hbm.at[:, pl.ds(my_pos * n_per, n_per)], w_ref, in_sems.at[1])
        w_dma.start()

        barrier_sem = pltpu.get_barrier_semaphore()
        for nbr in [left, right]:
            pl.semaphore_signal(
                barrier_sem, inc=1,
                device_id=(nbr,), device_id_type=pl.DeviceIdType.MESH,
            )
        pl.semaphore_wait(barrier_sem, 2)

        scale = s_ref[0, 0]
        copies = [None] * (N_DEV * N_Q)
        unit = [0]

        def compute_q(origin, slot_idx, qi):
            c = unit[0]
            unit[0] += 1
            yslot = c % N_YSLOTS
            if c >= N_YSLOTS:
                copies[c - N_YSLOTS].wait()
            chunk = comm_ref[slot_idx, qi * q:(qi + 1) * q, :]
            for j in range(N_TILES):
                acc = lax.dot_general(
                    chunk, w_ref[:, j * n_tile:(j + 1) * n_tile],
                    (((1,), (0,)), ((), ())),
                    preferred_element_type=jnp.int32,
                )
                y_ref[yslot, :, j * n_tile:(j + 1) * n_tile] = jnp.maximum(
                    acc.astype(jnp.float32) * scale, 0.0)
            cp = pltpu.make_async_copy(
                y_ref.at[yslot],
                out_ref.at[pl.ds(origin * m_per + qi * q, q), :],
                copy_sems.at[c],
            )
            cp.start()
            copies[c] = cp

        def xfer(src_slot, dst_slot, qi, sem_idx, target):
            return pltpu.make_async_remote_copy(
                src_ref=comm_ref.at[src_slot, pl.ds(qi * q, q)],
                dst_ref=comm_ref.at[dst_slot, pl.ds(qi * q, q)],
                send_sem=send_sems.at[sem_idx],
                recv_sem=recv_sems.at[sem_idx],
                device_id=(target,),
                device_id_type=pl.DeviceIdType.MESH,
            )

        x_dma.wait()

        send_r = [xfer(OWN, FROM_L, qi, qi, right) for qi in range(N_Q)]
        send_l = [xfer(OWN, FROM_R, qi, N_Q + qi, left) for qi in range(N_Q)]
        for qi in range(N_Q):
            send_r[qi].start()
            send_l[N_Q - 1 - qi].start()

        w_dma.wait()
        for qi in range(N_Q):
            compute_q(my_pos, OWN, qi)

        relay = {}

        def on_left(qi, rel):
            send_r[qi].wait_recv()
            if rel:
                relay[qi] = xfer(FROM_L, DIAG, qi, 8 + qi, right)
                relay[qi].start()
            compute_q(left, FROM_L, qi)

        def on_right(qi, rel):
            send_l[qi].wait_recv()
            if rel:
                relay[qi] = xfer(FROM_R, DIAG, qi, 8 + qi, left)
                relay[qi].start()
            compute_q(right, FROM_R, qi)

        on_left(0, True)
        on_right(3, True)
        on_left(1, True)
        on_right(2, True)
        on_left(2, False)
        on_right(1, False)
        on_left(3, False)
        on_right(0, False)

        diag = (my_pos + 2) % N_DEV
        for qi in (0, 3, 1, 2):
            relay[qi].wait_recv()
            compute_q(diag, DIAG, qi)

        for rdma in send_r + send_l + [relay[qi] for qi in range(N_Q)]:
            rdma.wait_send()
        for c in range(N_DEV * N_Q - N_YSLOTS, N_DEV * N_Q):
            copies[c].wait()

    return pl.pallas_call(
        body,
        out_shape=jax.ShapeDtypeStruct((N_DEV * m_per, n_per), jnp.float32),
        in_specs=[
            pl.BlockSpec(memory_space=pl.ANY),
            pl.BlockSpec(memory_space=pl.ANY),
            pl.BlockSpec(memory_space=pltpu.SMEM),
        ],
        out_specs=pl.BlockSpec(memory_space=pl.ANY),
        scratch_shapes=[
            pltpu.VMEM((N_DEV, m_per, k), x.dtype),
            pltpu.VMEM((k, n_per), w_mat.dtype),
            pltpu.VMEM((N_YSLOTS, q, n_per), jnp.float32),
            pltpu.SemaphoreType.DMA((12,)),
            pltpu.SemaphoreType.DMA((12,)),
            pltpu.SemaphoreType.DMA((N_DEV * N_Q,)),
            pltpu.SemaphoreType.DMA((2,)),
        ],
        compiler_params=pltpu.CompilerParams(
            collective_id=0,
            vmem_limit_bytes=60 * 1024 * 1024,
        ),
    )(x, w_mat, s)
